# Initial kernel scaffold; baseline (speedup 1.0000x reference)
import jax
import jax.numpy as jnp
from jax.experimental import pallas as pl


def kernel(g_edge_index, in_feat, edge_weights, W_self_0, W_neigh_0, b_0, W_self_1, W_neigh_1, b_1, W_self_2, W_neigh_2, b_2):
    raise NotImplementedError("write your pallas kernel here")


# speedup vs baseline: 2.5510x; 2.5510x over previous
import functools

import jax
import jax.numpy as jnp
from jax import lax
from jax.experimental import pallas as pl
from jax.experimental.pallas import tpu as pltpu
from jax.experimental.pallas import tpu_sc as plsc

N = 10000
E = 320000
D = 128
NC = 2
NS = 16
NW = NC * NS
L = 16
NP = 10240
EP = 327680
K = 128
NCH = (EP // NW) // K
RPT = NP // NS
ZR = 128


@functools.lru_cache(maxsize=None)
def _get_mesh():
  return plsc.VectorSubcoreMesh(
      core_axis_name="c", subcore_axis_name="s", num_cores=NC, num_subcores=NS)


def _seg_body(with_deg, edge_hbm, ew_hbm, p_hbm, *rest):
  if with_deg:
    (agg_out, deg_out, cmb_c, src_c, dst_c, ew_c, rows_v, ones_c, zd_c,
     agg_s, deg_s) = rest
  else:
    agg_out, cmb_c, src_c, dst_c, ew_c, rows_v, agg_s = rest
  c = lax.axis_index("c")
  s = lax.axis_index("s")
  w = s * NC + c

  def _zrow(i, _):
    for j in range(D // L):
      rows_v[i, pl.ds(j * L, L)] = jnp.zeros((L,), jnp.float32)
    return 0
  lax.fori_loop(0, ZR, _zrow, 0)
  for r in range(RPT // ZR):
    pltpu.sync_copy(rows_v, agg_s.at[pl.ds(s * RPT + r * ZR, ZR), :])
  if with_deg:
    def _zo(i, _):
      zd_c[pl.ds(i * L, L)] = jnp.zeros((L,), jnp.float32)
      ones_c[pl.ds(i * L, L)] = jnp.ones((L,), jnp.float32)
      return 0
    lax.fori_loop(0, K // L, _zo, 0)
    for r in range(RPT // K):
      pltpu.sync_copy(zd_c, deg_s.at[pl.ds(s * RPT + r * K, K)])
  plsc.subcore_barrier()

  def _chunk(ch, _):
    pltpu.sync_copy(edge_hbm.at[w, ch], cmb_c)
    pltpu.sync_copy(ew_hbm.at[w, ch], ew_c)

    def _dec(g, _):
      v = cmb_c[0, pl.ds(g * L, L)]
      src_c[0, pl.ds(g * L, L)] = jnp.bitwise_and(v, 16383)
      dst_c[0, pl.ds(g * L, L)] = jnp.right_shift(v, 14)
      return 0
    lax.fori_loop(0, K // L, _dec, 0)

    pltpu.sync_copy(p_hbm.at[src_c.at[0]], rows_v)

    def _grp(g, _):
      ewv = ew_c[0, pl.ds(g * L, L)]
      for e in range(L):
        sc = jnp.full((L,), 0.0, jnp.float32) + ewv[e]
        r = g * L + e
        for j in range(D // L):
          rows_v[r, pl.ds(j * L, L)] = rows_v[r, pl.ds(j * L, L)] * sc
      return 0
    lax.fori_loop(0, K // L, _grp, 0)

    pltpu.sync_copy(rows_v, agg_s.at[dst_c.at[0]], add=True)
    if with_deg:
      pltpu.sync_copy(ones_c, deg_s.at[dst_c.at[0]], add=True)
    return 0
  lax.fori_loop(0, NCH, _chunk, 0)
  plsc.subcore_barrier()

  pltpu.sync_copy(agg_s.at[pl.ds(s * RPT, RPT), :],
                  agg_out.at[c, pl.ds(s * RPT, RPT), :])
  if with_deg:
    pltpu.sync_copy(deg_s.at[pl.ds(s * RPT, RPT)],
                    deg_out.at[c, pl.ds(s * RPT, RPT)])


def _make_seg(with_deg):
  out_type = [jax.ShapeDtypeStruct((NC, NP, D), jnp.float32)]
  scratch = [
      pltpu.VMEM((1, K), jnp.int32),
      pltpu.VMEM((1, K), jnp.int32),
      pltpu.VMEM((1, K), jnp.int32),
      pltpu.VMEM((1, K), jnp.float32),
      pltpu.VMEM((K, D), jnp.float32),
  ]
  if with_deg:
    out_type.append(jax.ShapeDtypeStruct((NC, NP), jnp.float32))
    scratch += [
        pltpu.VMEM((K,), jnp.float32),
        pltpu.VMEM((K,), jnp.float32),
    ]
  scratch.append(pltpu.VMEM_SHARED((NP, D), jnp.float32))
  if with_deg:
    scratch.append(pltpu.VMEM_SHARED((NP,), jnp.float32))
  return pl.kernel(
      functools.partial(_seg_body, with_deg),
      out_type=out_type,
      mesh=_get_mesh(),
      scratch_types=scratch,
  )


_make_seg = functools.lru_cache(maxsize=None)(_make_seg)


BN = 1024


def _tc_pre_body(h_ref, wn_ref, ws_ref, p_ref, s_ref):
  h = h_ref[...]
  p_ref[...] = jnp.dot(h, wn_ref[...], preferred_element_type=jnp.float32)
  s_ref[...] = jnp.dot(h, ws_ref[...], preferred_element_type=jnp.float32)


_tc_pre = pl.pallas_call(
    _tc_pre_body,
    grid=(NP // BN,),
    in_specs=[
        pl.BlockSpec((BN, D), lambda i: (i, 0)),
        pl.BlockSpec((D, D), lambda i: (0, 0)),
        pl.BlockSpec((D, D), lambda i: (0, 0)),
    ],
    out_specs=[
        pl.BlockSpec((BN, D), lambda i: (i, 0)),
        pl.BlockSpec((BN, D), lambda i: (i, 0)),
    ],
    out_shape=[
        jax.ShapeDtypeStruct((NP, D), jnp.float32),
        jax.ShapeDtypeStruct((NP, D), jnp.float32),
    ],
)


def _tc_mid_body(s_ref, agg_ref, deg_ref, b_ref, wn_ref, ws_ref,
                 p_ref, s2_ref):
  a = agg_ref[0] + agg_ref[1]
  dg = deg_ref[0] + deg_ref[1]
  inv = 1.0 / jnp.maximum(dg, 1.0)
  h = s_ref[...] + a * inv + b_ref[...]
  h = jnp.maximum(h, 0.0)
  p_ref[...] = jnp.dot(h, wn_ref[...], preferred_element_type=jnp.float32)
  s2_ref[...] = jnp.dot(h, ws_ref[...], preferred_element_type=jnp.float32)


_tc_mid = pl.pallas_call(
    _tc_mid_body,
    grid=(NP // BN,),
    in_specs=[
        pl.BlockSpec((BN, D), lambda i: (i, 0)),
        pl.BlockSpec((NC, BN, D), lambda i: (0, i, 0)),
        pl.BlockSpec((NC, BN, 1), lambda i: (0, i, 0)),
        pl.BlockSpec((1, D), lambda i: (0, 0)),
        pl.BlockSpec((D, D), lambda i: (0, 0)),
        pl.BlockSpec((D, D), lambda i: (0, 0)),
    ],
    out_specs=[
        pl.BlockSpec((BN, D), lambda i: (i, 0)),
        pl.BlockSpec((BN, D), lambda i: (i, 0)),
    ],
    out_shape=[
        jax.ShapeDtypeStruct((NP, D), jnp.float32),
        jax.ShapeDtypeStruct((NP, D), jnp.float32),
    ],
)


def _tc_fin_body(s_ref, agg_ref, deg_ref, b_ref, o_ref):
  a = agg_ref[0] + agg_ref[1]
  dg = deg_ref[0] + deg_ref[1]
  inv = 1.0 / jnp.maximum(dg, 1.0)
  o_ref[...] = s_ref[...] + a * inv + b_ref[...]


_tc_fin = pl.pallas_call(
    _tc_fin_body,
    grid=(NP // BN,),
    in_specs=[
        pl.BlockSpec((BN, D), lambda i: (i, 0)),
        pl.BlockSpec((NC, BN, D), lambda i: (0, i, 0)),
        pl.BlockSpec((NC, BN, 1), lambda i: (0, i, 0)),
        pl.BlockSpec((1, D), lambda i: (0, 0)),
    ],
    out_specs=pl.BlockSpec((BN, D), lambda i: (i, 0)),
    out_shape=jax.ShapeDtypeStruct((NP, D), jnp.float32),
)


def kernel(g_edge_index, in_feat, edge_weights,
           W_self_0, W_neigh_0, b_0,
           W_self_1, W_neigh_1, b_1,
           W_self_2, W_neigh_2, b_2):
  npad = EP - E
  src = jnp.concatenate([g_edge_index[0], jnp.zeros((npad,), jnp.int32)])
  dst = jnp.concatenate(
      [g_edge_index[1], N + (jnp.arange(npad, dtype=jnp.int32) % (NP - N))])
  ew = jnp.concatenate([edge_weights, jnp.zeros((npad,), jnp.float32)])
  edge_r = (jnp.left_shift(dst, 14) | src).reshape(NW, NCH, 1, K)
  ew_r = ew.reshape(NW, NCH, 1, K)

  h0 = jnp.pad(in_feat, ((0, NP - N), (0, 0)))
  b0 = b_0.reshape(1, D)
  b1 = b_1.reshape(1, D)
  b2 = b_2.reshape(1, D)

  p0, s0 = _tc_pre(h0, W_neigh_0, W_self_0)
  agg0, deg = _make_seg(True)(edge_r, ew_r, p0)
  deg3 = deg[..., None]
  p1, s1 = _tc_mid(s0, agg0, deg3, b0, W_neigh_1, W_self_1)
  agg1, = _make_seg(False)(edge_r, ew_r, p1)
  p2, s2 = _tc_mid(s1, agg1, deg3, b1, W_neigh_2, W_self_2)
  agg2, = _make_seg(False)(edge_r, ew_r, p2)
  out = _tc_fin(s2, agg2, deg3, b2)
  return out[:N]


# speedup vs baseline: 3.4308x; 1.3449x over previous
import functools

import jax
import jax.numpy as jnp
from jax import lax
from jax.experimental import pallas as pl
from jax.experimental.pallas import tpu as pltpu
from jax.experimental.pallas import tpu_sc as plsc

N = 10000
E = 320000
D = 128
NC = 2
NS = 16
NW = NC * NS
L = 16
NP = 10240
EP = 327680
K = 128
NCH = (EP // NW) // K
RPT = NP // NS
ZR = 128


@functools.lru_cache(maxsize=None)
def _get_mesh():
  return plsc.VectorSubcoreMesh(
      core_axis_name="c", subcore_axis_name="s", num_cores=NC, num_subcores=NS)


def _seg_body(with_deg, edge_hbm, ew_hbm, p_hbm, *rest):
  if with_deg:
    (agg_out, deg_out, cmb2, src2, dst2, ew2, rows2, ones_c, zd_c,
     si2, sg2, ss2, sd2, agg_s, deg_s) = rest
  else:
    agg_out, cmb2, src2, dst2, ew2, rows2, si2, sg2, ss2, agg_s = rest
  c = lax.axis_index("c")
  s = lax.axis_index("s")
  w = s * NC + c

  def _zrow(i, _):
    for j in range(D // L):
      rows2[0, i, pl.ds(j * L, L)] = jnp.zeros((L,), jnp.float32)
    return 0
  lax.fori_loop(0, ZR, _zrow, 0)
  for r in range(RPT // ZR):
    pltpu.sync_copy(rows2.at[0], agg_s.at[pl.ds(s * RPT + r * ZR, ZR), :])
  if with_deg:
    def _zo(i, _):
      zd_c[pl.ds(i * L, L)] = jnp.zeros((L,), jnp.float32)
      ones_c[pl.ds(i * L, L)] = jnp.ones((L,), jnp.float32)
      return 0
    lax.fori_loop(0, K // L, _zo, 0)
    for r in range(RPT // K):
      pltpu.sync_copy(zd_c, deg_s.at[pl.ds(s * RPT + r * K, K)])
  plsc.subcore_barrier()

  def _start_cmb(i, t):
    pltpu.async_copy(edge_hbm.at[w, i], cmb2.at[t], si2.at[t])

  def _start_ew(i, t):
    pltpu.async_copy(ew_hbm.at[w, i], ew2.at[t], si2.at[t])

  def _wait_idx(i, t):
    pltpu.make_async_copy(edge_hbm.at[w, i], cmb2.at[t], si2.at[t]).wait()
    pltpu.make_async_copy(ew_hbm.at[w, i], ew2.at[t], si2.at[t]).wait()

  def _decode(t):
    def _dec(g, _):
      v = cmb2[t, 0, pl.ds(g * L, L)]
      src2[t, 0, pl.ds(g * L, L)] = jnp.bitwise_and(v, 16383)
      dst2[t, 0, pl.ds(g * L, L)] = jnp.right_shift(v, 14)
      return 0
    lax.fori_loop(0, K // L, _dec, 0)

  def _start_gather(t):
    pltpu.async_copy(p_hbm.at[src2.at[t, 0]], rows2.at[t], sg2.at[t])

  def _wait_gather(t):
    pltpu.make_async_copy(p_hbm.at[src2.at[t, 0]], rows2.at[t],
                          sg2.at[t]).wait()

  def _multiply(t):
    def _grp(g, _):
      ewv = ew2[t, 0, pl.ds(g * L, L)]
      for e in range(L):
        sc = jnp.full((L,), 0.0, jnp.float32) + ewv[e]
        r = g * L + e
        for j in range(D // L):
          rows2[t, r, pl.ds(j * L, L)] = rows2[t, r, pl.ds(j * L, L)] * sc
      return 0
    lax.fori_loop(0, K // L, _grp, 0)

  def _start_scatter(t):
    pltpu.async_copy(rows2.at[t], agg_s.at[dst2.at[t, 0]], ss2.at[t],
                     add=True)
    if with_deg:
      pltpu.async_copy(ones_c, deg_s.at[dst2.at[t, 0]], sd2.at[t], add=True)

  def _wait_scatter(t):
    pltpu.make_async_copy(rows2.at[t], agg_s.at[dst2.at[t, 0]],
                          ss2.at[t]).wait()
    if with_deg:
      pltpu.make_async_copy(ones_c, deg_s.at[dst2.at[t, 0]],
                            sd2.at[t]).wait()

  _start_cmb(0, 0)
  _start_ew(0, 0)

  def _half(i, t):
    @pl.when(i >= 2)
    def _():
      _wait_scatter(t)
    _wait_idx(i, t)
    _decode(t)
    _start_gather(t)
    @pl.when(i + 1 < NCH)
    def _():
      _start_cmb(i + 1, 1 - t)
    @pl.when(i >= 1)
    def _():
      _wait_gather(1 - t)
      _multiply(1 - t)
      _start_scatter(1 - t)
    @pl.when(i + 1 < NCH)
    def _():
      _start_ew(i + 1, 1 - t)

  def _pair(k, _):
    _half(2 * k, 0)
    _half(2 * k + 1, 1)
    return 0
  lax.fori_loop(0, NCH // 2, _pair, 0)

  _wait_gather(1)
  _multiply(1)
  _start_scatter(1)
  _wait_scatter(0)
  _wait_scatter(1)
  plsc.subcore_barrier()

  pltpu.sync_copy(agg_s.at[pl.ds(s * RPT, RPT), :],
                  agg_out.at[c, pl.ds(s * RPT, RPT), :])
  if with_deg:
    pltpu.sync_copy(deg_s.at[pl.ds(s * RPT, RPT)],
                    deg_out.at[c, pl.ds(s * RPT, RPT)])


def _make_seg(with_deg):
  out_type = [jax.ShapeDtypeStruct((NC, NP, D), jnp.float32)]
  scratch = [
      pltpu.VMEM((2, 1, K), jnp.int32),
      pltpu.VMEM((2, 1, K), jnp.int32),
      pltpu.VMEM((2, 1, K), jnp.int32),
      pltpu.VMEM((2, 1, K), jnp.float32),
      pltpu.VMEM((2, K, D), jnp.float32),
  ]
  if with_deg:
    out_type.append(jax.ShapeDtypeStruct((NC, NP), jnp.float32))
    scratch += [
        pltpu.VMEM((K,), jnp.float32),
        pltpu.VMEM((K,), jnp.float32),
    ]
  scratch += [
      pltpu.SemaphoreType.DMA((2,)),
      pltpu.SemaphoreType.DMA((2,)),
      pltpu.SemaphoreType.DMA((2,)),
  ]
  if with_deg:
    scratch.append(pltpu.SemaphoreType.DMA((2,)))
  scratch.append(pltpu.VMEM_SHARED((NP, D), jnp.float32))
  if with_deg:
    scratch.append(pltpu.VMEM_SHARED((NP,), jnp.float32))
  return pl.kernel(
      functools.partial(_seg_body, with_deg),
      out_type=out_type,
      mesh=_get_mesh(),
      scratch_types=scratch,
  )


_make_seg = functools.lru_cache(maxsize=None)(_make_seg)


BN = 1024


def _tc_pre_body(h_ref, wn_ref, ws_ref, p_ref, s_ref):
  h = h_ref[...]
  p_ref[...] = jnp.dot(h, wn_ref[...], preferred_element_type=jnp.float32)
  s_ref[...] = jnp.dot(h, ws_ref[...], preferred_element_type=jnp.float32)


_tc_pre = pl.pallas_call(
    _tc_pre_body,
    grid=(NP // BN,),
    in_specs=[
        pl.BlockSpec((BN, D), lambda i: (i, 0)),
        pl.BlockSpec((D, D), lambda i: (0, 0)),
        pl.BlockSpec((D, D), lambda i: (0, 0)),
    ],
    out_specs=[
        pl.BlockSpec((BN, D), lambda i: (i, 0)),
        pl.BlockSpec((BN, D), lambda i: (i, 0)),
    ],
    out_shape=[
        jax.ShapeDtypeStruct((NP, D), jnp.float32),
        jax.ShapeDtypeStruct((NP, D), jnp.float32),
    ],
)


def _tc_mid_body(s_ref, agg_ref, deg_ref, b_ref, wn_ref, ws_ref,
                 p_ref, s2_ref):
  a = agg_ref[0] + agg_ref[1]
  dg = deg_ref[0] + deg_ref[1]
  inv = 1.0 / jnp.maximum(dg, 1.0)
  h = s_ref[...] + a * inv + b_ref[...]
  h = jnp.maximum(h, 0.0)
  p_ref[...] = jnp.dot(h, wn_ref[...], preferred_element_type=jnp.float32)
  s2_ref[...] = jnp.dot(h, ws_ref[...], preferred_element_type=jnp.float32)


_tc_mid = pl.pallas_call(
    _tc_mid_body,
    grid=(NP // BN,),
    in_specs=[
        pl.BlockSpec((BN, D), lambda i: (i, 0)),
        pl.BlockSpec((NC, BN, D), lambda i: (0, i, 0)),
        pl.BlockSpec((NC, BN, 1), lambda i: (0, i, 0)),
        pl.BlockSpec((1, D), lambda i: (0, 0)),
        pl.BlockSpec((D, D), lambda i: (0, 0)),
        pl.BlockSpec((D, D), lambda i: (0, 0)),
    ],
    out_specs=[
        pl.BlockSpec((BN, D), lambda i: (i, 0)),
        pl.BlockSpec((BN, D), lambda i: (i, 0)),
    ],
    out_shape=[
        jax.ShapeDtypeStruct((NP, D), jnp.float32),
        jax.ShapeDtypeStruct((NP, D), jnp.float32),
    ],
)


def _tc_fin_body(s_ref, agg_ref, deg_ref, b_ref, o_ref):
  a = agg_ref[0] + agg_ref[1]
  dg = deg_ref[0] + deg_ref[1]
  inv = 1.0 / jnp.maximum(dg, 1.0)
  o_ref[...] = s_ref[...] + a * inv + b_ref[...]


_tc_fin = pl.pallas_call(
    _tc_fin_body,
    grid=(NP // BN,),
    in_specs=[
        pl.BlockSpec((BN, D), lambda i: (i, 0)),
        pl.BlockSpec((NC, BN, D), lambda i: (0, i, 0)),
        pl.BlockSpec((NC, BN, 1), lambda i: (0, i, 0)),
        pl.BlockSpec((1, D), lambda i: (0, 0)),
    ],
    out_specs=pl.BlockSpec((BN, D), lambda i: (i, 0)),
    out_shape=jax.ShapeDtypeStruct((NP, D), jnp.float32),
)


def kernel(g_edge_index, in_feat, edge_weights,
           W_self_0, W_neigh_0, b_0,
           W_self_1, W_neigh_1, b_1,
           W_self_2, W_neigh_2, b_2):
  npad = EP - E
  src = jnp.concatenate([g_edge_index[0], jnp.zeros((npad,), jnp.int32)])
  dst = jnp.concatenate(
      [g_edge_index[1], N + (jnp.arange(npad, dtype=jnp.int32) % (NP - N))])
  ew = jnp.concatenate([edge_weights, jnp.zeros((npad,), jnp.float32)])
  edge_r = (jnp.left_shift(dst, 14) | src).reshape(NW, NCH, 1, K)
  ew_r = ew.reshape(NW, NCH, 1, K)

  h0 = jnp.pad(in_feat, ((0, NP - N), (0, 0)))
  b0 = b_0.reshape(1, D)
  b1 = b_1.reshape(1, D)
  b2 = b_2.reshape(1, D)

  p0, s0 = _tc_pre(h0, W_neigh_0, W_self_0)
  agg0, deg = _make_seg(True)(edge_r, ew_r, p0)
  deg3 = deg[..., None]
  p1, s1 = _tc_mid(s0, agg0, deg3, b0, W_neigh_1, W_self_1)
  agg1, = _make_seg(False)(edge_r, ew_r, p1)
  p2, s2 = _tc_mid(s1, agg1, deg3, b1, W_neigh_2, W_self_2)
  agg2, = _make_seg(False)(edge_r, ew_r, p2)
  out = _tc_fin(s2, agg2, deg3, b2)
  return out[:N]
